# baseline (speedup 1.0000x reference)
import jax
import jax.numpy as jnp
from jax.experimental import pallas as pl
from jax.experimental.pallas import tpu as pltpu


def _fused_upsample_project_kernel(x_ref, a_ref, o_ref):
    v = x_ref[0]
    ch, hdim, wdim = v.shape
    hh2 = hdim // 2
    wh = wdim // 2
    mu = jnp.mean(v, axis=(1, 2), keepdims=True)
    vc = (v - mu).astype(jnp.bfloat16).reshape(2 * ch, hh2, wdim)
    a = a_ref[...].astype(jnp.bfloat16)

    for g in (0, 1):
        f = jax.lax.dot_general(
            a, vc[:, :, g * wh:(g + 1) * wh],
            dimension_numbers=(((0,), (2,)), ((), ())),
            preferred_element_type=jnp.float32)
        vg = f.transpose(0, 2, 1).astype(o_ref.dtype)
        o_ref[0, :, g::4, :] = vg
        o_ref[0, :, g + 2::4, :] = vg


def kernel(x, weight):
    B, C, H, W = x.shape
    C_out = weight.shape[0]
    assert C % 2 == 0 and 2 * W == 2 * C and 2 * H == 2 * C and H % 16 == 0
    ch = C // 2
    wh = W // 2

    a_mat = (weight[:, 0::2] + weight[:, 1::2]).T.astype(x.dtype)

    return pl.pallas_call(
        _fused_upsample_project_kernel,
        out_shape=jax.ShapeDtypeStruct((B, C_out, 2 * W, 2 * H), x.dtype),
        grid=(B, 2),
        in_specs=[
            pl.BlockSpec((1, ch, H, W), lambda b, j: (b, j, 0, 0)),
            pl.BlockSpec((wh, C_out), lambda b, j: (0, 0)),
        ],
        out_specs=pl.BlockSpec((1, C_out, 2 * W, 2 * ch),
                               lambda b, j: (b, 0, 0, j)),
        compiler_params=pltpu.CompilerParams(
            dimension_semantics=("parallel", "parallel"),
            vmem_limit_bytes=48 * 1024 * 1024),
    )(x, a_mat)


# speedup vs baseline: 1.0234x; 1.0234x over previous
import jax
import jax.numpy as jnp
from jax.experimental import pallas as pl
from jax.experimental.pallas import tpu as pltpu


def _fused_upsample_project_kernel(x_ref, a_ref, o_ref):
    v = x_ref[0]
    ch, hdim, wdim = v.shape
    hh2 = hdim // 2
    wh = wdim // 2
    mu = jnp.mean(v, axis=(1, 2), keepdims=True)
    vc = (v - mu).reshape(2 * ch, hh2, wdim)
    a = a_ref[...]

    for g in (0, 1):
        f = jax.lax.dot_general(
            a, vc[:, :, g * wh:(g + 1) * wh],
            dimension_numbers=(((0,), (2,)), ((), ())),
            preferred_element_type=jnp.float32)
        vg = f.transpose(0, 2, 1).astype(o_ref.dtype)
        o_ref[0, :, g::4, :] = vg
        o_ref[0, :, g + 2::4, :] = vg


def kernel(x, weight):
    B, C, H, W = x.shape
    C_out = weight.shape[0]
    assert C % 2 == 0 and 2 * W == 2 * C and 2 * H == 2 * C and H % 16 == 0
    ch = C // 2
    wh = W // 2

    a_mat = (weight[:, 0::2] + weight[:, 1::2]).T.astype(x.dtype)

    return pl.pallas_call(
        _fused_upsample_project_kernel,
        out_shape=jax.ShapeDtypeStruct((B, C_out, 2 * W, 2 * H), x.dtype),
        grid=(B, 2),
        in_specs=[
            pl.BlockSpec((1, ch, H, W), lambda b, j: (b, j, 0, 0)),
            pl.BlockSpec((wh, C_out), lambda b, j: (0, 0)),
        ],
        out_specs=pl.BlockSpec((1, C_out, 2 * W, 2 * ch),
                               lambda b, j: (b, 0, 0, j)),
        compiler_params=pltpu.CompilerParams(
            dimension_semantics=("parallel", "parallel"),
            vmem_limit_bytes=48 * 1024 * 1024),
    )(x, a_mat)
